# Initial kernel scaffold; baseline (speedup 1.0000x reference)
import jax
import jax.numpy as jnp
from jax.experimental import pallas as pl
from jax.experimental.pallas import tpu as pltpu

_K = 20
_TEMP = 0.07
_NUM_CLASSES = 1000
_W = 512
_SLOTS = 128
_CPAD = 1024


def _knn_kernel(feat_ref, qf_ref, qlab_ref, lab_ref, out_ref,
                s_ref, tv_ref, tl_ref):
    n = feat_ref.shape[0]
    w = qf_ref.shape[0]
    num_chunks = pl.num_programs(0)
    j = pl.program_id(0)

    slot_iota = jax.lax.broadcasted_iota(jnp.int32, (n, _SLOTS), 1)
    lane_iota = jax.lax.broadcasted_iota(jnp.int32, (n, w), 1)

    @pl.when(j == 0)
    def _init():
        tv_ref[...] = jnp.where(slot_iota < _K,
                                -jnp.inf, jnp.inf).astype(jnp.float32)
        tl_ref[...] = jnp.zeros((n, _SLOTS), jnp.int32)

    q_bf = qf_ref[...].astype(jnp.bfloat16)
    s = jax.lax.dot_general(feat_ref[...], q_bf, (((1,), (1,)), ((), ())),
                            preferred_element_type=jnp.float32)
    s_ref[...] = s
    qlab = qlab_ref[0]

    def _cond(carry):
        return carry[0]

    def _body(carry):
        _, m = carry
        tvv = tv_ref[...]
        thr = jnp.min(tvv, axis=1, keepdims=True)
        upd = m > thr
        sv = s_ref[...]
        pos = jnp.min(jnp.where(sv == m, lane_iota, w),
                      axis=1, keepdims=True)
        sel = lane_iota == pos
        lab = jnp.sum(jnp.where(sel, qlab, 0), axis=1, keepdims=True)
        sv = jnp.where(sel, -jnp.inf, sv)
        s_ref[...] = sv
        spos = jnp.min(jnp.where(tvv == thr, slot_iota, _SLOTS),
                       axis=1, keepdims=True)
        msk = (slot_iota == spos) & upd
        tvv = jnp.where(msk, m, tvv)
        tv_ref[...] = tvv
        tl_ref[...] = jnp.where(msk, lab, tl_ref[...])
        m2 = jnp.max(sv, axis=1, keepdims=True)
        thr2 = jnp.min(tvv, axis=1, keepdims=True)
        return jnp.any(m2 > thr2), m2

    m0 = jnp.max(s, axis=1, keepdims=True)
    thr0 = jnp.min(tv_ref[...], axis=1, keepdims=True)
    jax.lax.while_loop(_cond, _body, (jnp.any(m0 > thr0), m0))

    @pl.when(j == num_chunks - 1)
    def _final():
        cls_iota = jax.lax.broadcasted_iota(jnp.int32, (n, _CPAD), 1)
        scores = jnp.zeros((n, _CPAD), jnp.float32)
        for s_idx in range(_K):
            v = tv_ref[:, s_idx:s_idx + 1]
            l = tl_ref[:, s_idx:s_idx + 1]
            wgt = jnp.exp(v / jnp.float32(_TEMP))
            scores = scores + jnp.where(cls_iota == l, wgt, 0.0)
        mx = jnp.max(scores, axis=1, keepdims=True)
        pred = jnp.min(jnp.where(scores == mx, cls_iota, _CPAD),
                       axis=1, keepdims=True)
        correct = (pred == lab_ref[...]).astype(jnp.float32)
        out_ref[0, 0] = jnp.sum(correct) / jnp.float32(n)


def kernel(features, labels, queue_features, queue_labels):
    n, d = features.shape
    qs = queue_features.shape[0]
    w = min(_W, qs)
    c = qs // w
    feat_bf = features.astype(jnp.bfloat16)
    qlab3 = queue_labels.reshape(c, 1, w)
    lab2 = labels.reshape(n, 1)
    out = pl.pallas_call(
        _knn_kernel,
        grid=(c,),
        in_specs=[
            pl.BlockSpec((n, d), lambda j: (0, 0)),
            pl.BlockSpec((w, d), lambda j: (j, 0)),
            pl.BlockSpec((1, 1, w), lambda j: (j, 0, 0)),
            pl.BlockSpec((n, 1), lambda j: (0, 0)),
        ],
        out_specs=pl.BlockSpec((1, 1), lambda j: (0, 0)),
        out_shape=jax.ShapeDtypeStruct((1, 1), jnp.float32),
        scratch_shapes=[
            pltpu.VMEM((n, w), jnp.float32),
            pltpu.VMEM((n, _SLOTS), jnp.float32),
            pltpu.VMEM((n, _SLOTS), jnp.int32),
        ],
        compiler_params=pltpu.CompilerParams(
            dimension_semantics=("arbitrary",)),
    )(feat_bf, queue_features, qlab3, lab2)
    return out[0, 0]


# speedup vs baseline: 3.8090x; 3.8090x over previous
import jax
import jax.numpy as jnp
from jax.experimental import pallas as pl
from jax.experimental.pallas import tpu as pltpu

_K = 20
_TEMP = 0.07
_NUM_CLASSES = 1000
_W = 512
_SLOTS = 128
_CPAD = 1024


def _knn_kernel(feat_ref, qf_ref, qlab_ref, lab_ref, out_ref,
                s_ref, tv_ref, tl_ref):
    n = feat_ref.shape[0]
    w = qf_ref.shape[0]
    num_chunks = pl.num_programs(0)
    j = pl.program_id(0)

    slot_iota = jax.lax.broadcasted_iota(jnp.int32, (n, _SLOTS), 1)
    lane_iota = jax.lax.broadcasted_iota(jnp.int32, (n, w), 1)

    @pl.when(j == 0)
    def _init():
        tv_ref[...] = jnp.where(slot_iota < _K,
                                -jnp.inf, jnp.inf).astype(jnp.float32)
        tl_ref[...] = jnp.zeros((n, _SLOTS), jnp.int32)

    q_bf = qf_ref[...].astype(jnp.bfloat16)
    s = jax.lax.dot_general(feat_ref[...], q_bf, (((1,), (1,)), ((), ())),
                            preferred_element_type=jnp.float32)
    s_ref[...] = s
    qlab = qlab_ref[0]

    def _cond(carry):
        return carry[0]

    def _body(carry):
        _, m = carry
        tvv = tv_ref[...]
        thr = jnp.min(tvv, axis=1, keepdims=True)
        upd = m > thr
        sv = s_ref[...]
        pos = jnp.min(jnp.where(sv == m, lane_iota, w),
                      axis=1, keepdims=True)
        sel = lane_iota == pos
        lab = jnp.sum(jnp.where(sel, qlab, 0), axis=1, keepdims=True)
        sv = jnp.where(sel, -jnp.inf, sv)
        s_ref[...] = sv
        spos = jnp.min(jnp.where(tvv == thr, slot_iota, _SLOTS),
                       axis=1, keepdims=True)
        msk = (slot_iota == spos) & upd
        tvv = jnp.where(msk, m, tvv)
        tv_ref[...] = tvv
        tl_ref[...] = jnp.where(msk, lab, tl_ref[...])
        m2 = jnp.max(sv, axis=1, keepdims=True)
        thr2 = jnp.min(tvv, axis=1, keepdims=True)
        return jnp.any(m2 > thr2), m2

    m0 = jnp.max(s, axis=1, keepdims=True)
    thr0 = jnp.min(tv_ref[...], axis=1, keepdims=True)
    jax.lax.while_loop(_cond, _body, (jnp.any(m0 > thr0), m0))

    @pl.when(j == num_chunks - 1)
    def _final():
        cls_iota = jax.lax.broadcasted_iota(jnp.int32, (n, _CPAD), 1)
        scores = jnp.zeros((n, _CPAD), jnp.float32)
        for s_idx in range(_K):
            v = tv_ref[:, s_idx:s_idx + 1]
            l = tl_ref[:, s_idx:s_idx + 1]
            wgt = jnp.exp(v / jnp.float32(_TEMP))
            scores = scores + jnp.where(cls_iota == l, wgt, 0.0)
        mx = jnp.max(scores, axis=1, keepdims=True)
        pred = jnp.min(jnp.where(scores == mx, cls_iota, _CPAD),
                       axis=1, keepdims=True)
        correct = (pred == lab_ref[...]).astype(jnp.float32)
        out_ref[...] = jnp.sum(correct, keepdims=True) / jnp.float32(n)


def kernel(features, labels, queue_features, queue_labels):
    n, d = features.shape
    qs = queue_features.shape[0]
    w = min(_W, qs)
    c = qs // w
    feat_bf = features.astype(jnp.bfloat16)
    qlab3 = queue_labels.reshape(c, 1, w)
    lab2 = labels.reshape(n, 1)
    out = pl.pallas_call(
        _knn_kernel,
        grid=(c,),
        in_specs=[
            pl.BlockSpec((n, d), lambda j: (0, 0)),
            pl.BlockSpec((w, d), lambda j: (j, 0)),
            pl.BlockSpec((1, 1, w), lambda j: (j, 0, 0)),
            pl.BlockSpec((n, 1), lambda j: (0, 0)),
        ],
        out_specs=pl.BlockSpec((1, 1), lambda j: (0, 0)),
        out_shape=jax.ShapeDtypeStruct((1, 1), jnp.float32),
        scratch_shapes=[
            pltpu.VMEM((n, w), jnp.float32),
            pltpu.VMEM((n, _SLOTS), jnp.float32),
            pltpu.VMEM((n, _SLOTS), jnp.int32),
        ],
        compiler_params=pltpu.CompilerParams(
            dimension_semantics=("arbitrary",)),
    )(feat_bf, queue_features, qlab3, lab2)
    return out[0, 0]
